# baseline (speedup 1.0000x reference)
import jax
import jax.numpy as jnp
from jax.experimental import pallas as pl
from jax.experimental.pallas import tpu as pltpu

TOKENS = 32768
D_MODEL = 1024
NUM_EXPERTS = 64
THRESHOLD = 0.5
TILE = 2048


def _gate_kernel(x_ref, wt_ref, b_ref, gated_ref, mask_ref):
    logits = jnp.dot(x_ref[...], wt_ref[...], preferred_element_type=jnp.float32)
    logits = logits + b_ref[...]
    probs = jax.nn.sigmoid(logits)
    mask = (probs > THRESHOLD).astype(jnp.float32)
    gated_ref[...] = probs * mask
    mask_ref[...] = mask


def kernel(x, W, b):
    wt = W.T
    b2 = b.reshape(1, NUM_EXPERTS)
    grid = (TOKENS // TILE,)
    out_shape = jax.ShapeDtypeStruct((TOKENS, NUM_EXPERTS), jnp.float32)
    gated, mask = pl.pallas_call(
        _gate_kernel,
        grid=grid,
        in_specs=[
            pl.BlockSpec((TILE, D_MODEL), lambda i: (i, 0)),
            pl.BlockSpec((D_MODEL, NUM_EXPERTS), lambda i: (0, 0)),
            pl.BlockSpec((1, NUM_EXPERTS), lambda i: (0, 0)),
        ],
        out_specs=[
            pl.BlockSpec((TILE, NUM_EXPERTS), lambda i: (i, 0)),
            pl.BlockSpec((TILE, NUM_EXPERTS), lambda i: (i, 0)),
        ],
        out_shape=[out_shape, out_shape],
        compiler_params=pltpu.CompilerParams(
            dimension_semantics=("arbitrary",),
        ),
    )(x, wt, b2)
    return gated, mask
